# Initial kernel scaffold; baseline (speedup 1.0000x reference)
import jax
import jax.numpy as jnp
from jax.experimental import pallas as pl


def kernel(edge_index, edge_feats, node_scales, sink_node_mask, W, b0):
    raise NotImplementedError("write your pallas kernel here")


# speedup vs baseline: 55.9692x; 55.9692x over previous
import functools

import jax
import jax.numpy as jnp
from jax import lax
from jax.experimental import pallas as pl
from jax.experimental.pallas import tpu as pltpu
from jax.experimental.pallas import tpu_sc as plsc

N = 100000
E = 3200000
EDGE_DIM = 16
N_ITERS = 12

NCORES = 2
NSUB = 16
NTILES = NCORES * NSUB
LN2 = 0.6931471805599453

NPAD = 101376
SLICE = NPAD // NSUB
VSLICE = NPAD // NTILES
SCH = 2112
NSCH = NPAD // SCH
EPT = E // NTILES
CH = 2000
NCH = EPT // CH
VPC = CH // 16

_mesh = plsc.VectorSubcoreMesh(core_axis_name="c", subcore_axis_name="s")
_sc_params = pltpu.CompilerParams(needs_layout_passes=False)


def _vlog(xv):
    xv = jnp.maximum(xv, jnp.float32(1e-12))
    bits = plsc.bitcast(xv, jnp.int32)
    e = (bits >> 23) - 127
    m = plsc.bitcast((bits & 0x7FFFFF) | 0x3F800000, jnp.float32)
    big = m > jnp.float32(1.4142135)
    m = jnp.where(big, m * jnp.float32(0.5), m)
    ef = (e + big.astype(jnp.int32)).astype(jnp.float32)
    s = (m - 1.0) / (m + 1.0)
    s2 = s * s
    p = jnp.float32(2.0 / 9.0)
    p = p * s2 + jnp.float32(2.0 / 7.0)
    p = p * s2 + jnp.float32(2.0 / 5.0)
    p = p * s2 + jnp.float32(2.0 / 3.0)
    p = p * s2 + jnp.float32(2.0)
    return ef * jnp.float32(LN2) + p * s


def _vpow(xv, pv):
    return jnp.exp(pv * _vlog(xv))


def _rewards_body(ef_ref, wb_ref, b0_ref, out_ref):
    enc = jnp.dot(ef_ref[...], wb_ref[...],
                  preferred_element_type=jnp.float32) + b0_ref[0]
    out_ref[...] = -jnp.logaddexp(enc, 0.0)


def _rewards_tc(edge_feats, W, b0):
    rows = E // 8
    blk = 4000
    ef2 = edge_feats.reshape(rows, 128)
    wb = jnp.kron(jnp.eye(8, dtype=jnp.float32), W)
    out = pl.pallas_call(
        _rewards_body,
        grid=(rows // blk,),
        in_specs=[
            pl.BlockSpec((blk, 128), lambda i: (i, 0)),
            pl.BlockSpec((128, 8), lambda i: (0, 0)),
            pl.BlockSpec(memory_space=pltpu.SMEM),
        ],
        out_specs=pl.BlockSpec((blk, 8), lambda i: (i, 0)),
        out_shape=jax.ShapeDtypeStruct((rows, 8), jnp.float32),
    )(ef2, wb, b0)
    return out.reshape(E)


def _prep_body(src_hbm, dst_hbm, rew_hbm, mu_hbm, m_hbm, pw_hbm,
               mu_l, sbuf, dbuf, rbuf, mbuf, pbuf):
    cid = lax.axis_index("c")
    sid = lax.axis_index("s")
    wid = cid * NSUB + sid
    pltpu.sync_copy(mu_hbm, mu_l)
    ebase = wid * EPT

    def chunk(c, _):
        off = ebase + c * CH
        pltpu.sync_copy(src_hbm.at[pl.ds(off, CH)], sbuf)
        pltpu.sync_copy(dst_hbm.at[pl.ds(off, CH)], dbuf)
        pltpu.sync_copy(rew_hbm.at[pl.ds(off, CH)], rbuf)

        def vec(i, _):
            sl = pl.ds(i * 16, 16)
            mu_j = plsc.load_gather(mu_l, [sbuf[sl]])
            mu_i = plsc.load_gather(mu_l, [dbuf[sl]])
            inv = 1.0 / mu_i
            mbuf[sl] = jnp.exp(rbuf[sl] * inv)
            pbuf[sl] = mu_j * inv
            return 0

        lax.fori_loop(0, VPC, vec, 0)
        pltpu.sync_copy(mbuf, m_hbm.at[pl.ds(off, CH)])
        pltpu.sync_copy(pbuf, pw_hbm.at[pl.ds(off, CH)])
        return 0

    lax.fori_loop(0, NCH, chunk, 0)


def _prep_sc(src, dst, rewards, mu_pad):
    return pl.kernel(
        _prep_body,
        out_type=(jax.ShapeDtypeStruct((E,), jnp.float32),
                  jax.ShapeDtypeStruct((E,), jnp.float32)),
        mesh=_mesh,
        compiler_params=_sc_params,
        scratch_types=[
            pltpu.VMEM((NPAD,), jnp.float32),
            pltpu.VMEM((CH,), jnp.int32),
            pltpu.VMEM((CH,), jnp.int32),
            pltpu.VMEM((CH,), jnp.float32),
            pltpu.VMEM((CH,), jnp.float32),
            pltpu.VMEM((CH,), jnp.float32),
        ],
    )(src, dst, rewards, mu_pad)


def _stage_x(p_hbm, b_hbm, x_l, bufa, bufb, bufc):
    def chunk(c, _):
        off = c * SCH
        pltpu.sync_copy(p_hbm.at[pl.ds(off, SCH)], bufa)
        pltpu.sync_copy(p_hbm.at[pl.ds(NPAD + off, SCH)], bufb)
        pltpu.sync_copy(b_hbm.at[pl.ds(off, SCH)], bufc)

        def vec(j, _):
            sl = pl.ds(j * 16, 16)
            x_l[pl.ds(off + j * 16, 16)] = bufa[sl] + bufb[sl] + bufc[sl]
            return 0

        lax.fori_loop(0, SCH // 16, vec, 0)
        return 0

    lax.fori_loop(0, NSCH, chunk, 0)


def _iter_body(p_hbm, b_hbm, src_hbm, dst_hbm, m_hbm, pw_hbm, pout_hbm,
               x_l, bufa, bufb, bufc, sbuf, dbuf, mbuf, pbuf, msg, zbuf, acc):
    cid = lax.axis_index("c")
    sid = lax.axis_index("s")
    wid = cid * NSUB + sid
    def zvec(j, _):
        zbuf[pl.ds(j * 16, 16)] = jnp.zeros((16,), jnp.float32)
        return 0

    lax.fori_loop(0, SLICE // 16, zvec, 0)
    pltpu.sync_copy(zbuf, acc.at[pl.ds(sid * SLICE, SLICE)])
    _stage_x(p_hbm, b_hbm, x_l, bufa, bufb, bufc)
    plsc.subcore_barrier()

    ebase = wid * EPT

    def chunk(c, _):
        off = ebase + c * CH
        pltpu.sync_copy(src_hbm.at[pl.ds(off, CH)], sbuf)
        pltpu.sync_copy(dst_hbm.at[pl.ds(off, CH)], dbuf)
        pltpu.sync_copy(m_hbm.at[pl.ds(off, CH)], mbuf)
        pltpu.sync_copy(pw_hbm.at[pl.ds(off, CH)], pbuf)

        def vec(i, _):
            sl = pl.ds(i * 16, 16)
            xs = plsc.load_gather(x_l, [sbuf[sl]])
            msg[sl] = mbuf[sl] * _vpow(xs, pbuf[sl])
            return 0

        lax.fori_loop(0, VPC, vec, 0)
        pltpu.sync_copy(msg, acc.at[dbuf], add=True)
        return 0

    lax.fori_loop(0, NCH, chunk, 0)
    plsc.subcore_barrier()
    pltpu.sync_copy(acc.at[pl.ds(sid * SLICE, SLICE)], zbuf)
    pltpu.sync_copy(zbuf,
                    pout_hbm.at[pl.ds(cid * NPAD + sid * SLICE, SLICE)])


def _iter_sc(p_prev, b_pad, src, dst, M, pw):
    return pl.kernel(
        _iter_body,
        out_type=jax.ShapeDtypeStruct((2 * NPAD,), jnp.float32),
        mesh=_mesh,
        compiler_params=_sc_params,
        scratch_types=[
            pltpu.VMEM((NPAD,), jnp.float32),
            pltpu.VMEM((SCH,), jnp.float32),
            pltpu.VMEM((SCH,), jnp.float32),
            pltpu.VMEM((SCH,), jnp.float32),
            pltpu.VMEM((CH,), jnp.int32),
            pltpu.VMEM((CH,), jnp.int32),
            pltpu.VMEM((CH,), jnp.float32),
            pltpu.VMEM((CH,), jnp.float32),
            pltpu.VMEM((CH,), jnp.float32),
            pltpu.VMEM((SLICE,), jnp.float32),
            pltpu.VMEM_SHARED((NPAD,), jnp.float32),
        ],
    )(p_prev, b_pad, src, dst, M, pw)


def _epi_body(p_hbm, b_hbm, src_hbm, dst_hbm, m_hbm, pw_hbm,
              val_hbm, ep_hbm,
              x_l, bufa, bufb, bufc, sbuf, dbuf, mbuf, pbuf, ebuf, vbuf):
    cid = lax.axis_index("c")
    sid = lax.axis_index("s")
    wid = cid * NSUB + sid
    _stage_x(p_hbm, b_hbm, x_l, bufa, bufb, bufc)

    vbase = wid * VSLICE

    def vvec(j, _):
        xv = x_l[pl.ds(vbase + j * 16, 16)]
        lv = _vlog(xv)
        lv = jnp.where(xv == 0.0,
                       jnp.full((16,), -jnp.inf, jnp.float32), lv)
        vbuf[pl.ds(j * 16, 16)] = lv
        return 0

    lax.fori_loop(0, VSLICE // 16, vvec, 0)
    pltpu.sync_copy(vbuf, val_hbm.at[pl.ds(vbase, VSLICE)])

    ebase = wid * EPT

    def chunk(c, _):
        off = ebase + c * CH
        pltpu.sync_copy(src_hbm.at[pl.ds(off, CH)], sbuf)
        pltpu.sync_copy(dst_hbm.at[pl.ds(off, CH)], dbuf)
        pltpu.sync_copy(m_hbm.at[pl.ds(off, CH)], mbuf)
        pltpu.sync_copy(pw_hbm.at[pl.ds(off, CH)], pbuf)

        def vec(i, _):
            sl = pl.ds(i * 16, 16)
            xs = plsc.load_gather(x_l, [sbuf[sl]])
            xd = plsc.load_gather(x_l, [dbuf[sl]])
            ebuf[sl] = mbuf[sl] * _vpow(xs, pbuf[sl]) / xd
            return 0

        lax.fori_loop(0, VPC, vec, 0)
        pltpu.sync_copy(ebuf, ep_hbm.at[pl.ds(off, CH)])
        return 0

    lax.fori_loop(0, NCH, chunk, 0)


def _epi_sc(p_last, b_pad, src, dst, M, pw):
    return pl.kernel(
        _epi_body,
        out_type=(jax.ShapeDtypeStruct((NPAD,), jnp.float32),
                  jax.ShapeDtypeStruct((E,), jnp.float32)),
        mesh=_mesh,
        compiler_params=_sc_params,
        scratch_types=[
            pltpu.VMEM((NPAD,), jnp.float32),
            pltpu.VMEM((SCH,), jnp.float32),
            pltpu.VMEM((SCH,), jnp.float32),
            pltpu.VMEM((SCH,), jnp.float32),
            pltpu.VMEM((CH,), jnp.int32),
            pltpu.VMEM((CH,), jnp.int32),
            pltpu.VMEM((CH,), jnp.float32),
            pltpu.VMEM((CH,), jnp.float32),
            pltpu.VMEM((CH,), jnp.float32),
            pltpu.VMEM((VSLICE,), jnp.float32),
        ],
    )(p_last, b_pad, src, dst, M, pw)


def kernel(edge_index, edge_feats, node_scales, sink_node_mask, W, b0):
    src = edge_index[0].astype(jnp.int32)
    dst = edge_index[1].astype(jnp.int32)
    pad = NPAD - N
    mu_pad = jnp.concatenate(
        [node_scales, jnp.ones((pad,), jnp.float32)])
    b_pad = jnp.concatenate(
        [sink_node_mask, jnp.zeros((pad,), jnp.float32)])

    rewards = _rewards_tc(edge_feats, W, b0)
    M, pw = _prep_sc(src, dst, rewards, mu_pad)

    p0 = jnp.zeros((2 * NPAD,), jnp.float32)
    p_last = lax.fori_loop(
        0, N_ITERS,
        lambda i, p: _iter_sc(p, b_pad, src, dst, M, pw),
        p0)

    values_pad, edge_probs = _epi_sc(p_last, b_pad, src, dst, M, pw)
    return rewards, values_pad[:N], edge_probs


# speedup vs baseline: 68.2975x; 1.2203x over previous
import jax
import jax.numpy as jnp
from jax import lax
from jax.experimental import pallas as pl
from jax.experimental.pallas import tpu as pltpu
from jax.experimental.pallas import tpu_sc as plsc

N = 100000
E = 3200000
N_ITERS = 12

NCORES = 2
NSUB = 16
NTILES = NCORES * NSUB
LN2 = 0.6931471805599453

NPAD = 100352
SLICE = NPAD // NSUB
VSLICE = NPAD // NTILES
SCH = 1568
NSCH = NPAD // SCH
EPT = E // NTILES
CH = 2000
NCH = EPT // CH
VPC = CH // 16

_mesh = plsc.VectorSubcoreMesh(core_axis_name="c", subcore_axis_name="s")
_sc_params = pltpu.CompilerParams(needs_layout_passes=False)


def _vlog(xv):
    xv = jnp.maximum(xv, jnp.float32(1e-12))
    bits = plsc.bitcast(xv, jnp.int32)
    e = (bits >> 23) - 127
    m = plsc.bitcast((bits & 0x7FFFFF) | 0x3F800000, jnp.float32)
    big = m > jnp.float32(1.4142135)
    m = jnp.where(big, m * jnp.float32(0.5), m)
    ef = (e + big.astype(jnp.int32)).astype(jnp.float32)
    s = (m - 1.0) / (m + 1.0)
    s2 = s * s
    p = jnp.float32(2.0 / 9.0)
    p = p * s2 + jnp.float32(2.0 / 7.0)
    p = p * s2 + jnp.float32(2.0 / 5.0)
    p = p * s2 + jnp.float32(2.0 / 3.0)
    p = p * s2 + jnp.float32(2.0)
    return ef * jnp.float32(LN2) + p * s


def _vpow(xv, pv):
    return jnp.exp(pv * _vlog(xv))


def _rewards_body(ef_ref, w_ref, b0_ref, out_ref):
    enc = jnp.dot(ef_ref[...], w_ref[...],
                  preferred_element_type=jnp.float32)[:, 0] + b0_ref[0]
    out_ref[...] = -jnp.logaddexp(enc, 0.0)


def _rewards_tc(edge_feats, W, b0):
    blk = 25600
    w_row = W
    return pl.pallas_call(
        _rewards_body,
        grid=(E // blk,),
        in_specs=[
            pl.BlockSpec((blk, 16), lambda i: (i, 0)),
            pl.BlockSpec((16, 1), lambda i: (0, 0)),
            pl.BlockSpec(memory_space=pltpu.SMEM),
        ],
        out_specs=pl.BlockSpec((blk,), lambda i: (i,)),
        out_shape=jax.ShapeDtypeStruct((E,), jnp.float32),
    )(edge_feats, w_row, b0)


def _in_start(hbm_refs, bufs, off, n, sem):
    for h, b in zip(hbm_refs, bufs):
        pltpu.async_copy(h.at[pl.ds(off, n)], b.at[pl.ds(0, n)], sem)


def _in_wait(hbm_refs, bufs, off, n, sem):
    for h, b in zip(hbm_refs, bufs):
        pltpu.make_async_copy(h.at[pl.ds(off, n)], b.at[pl.ds(0, n)],
                              sem).wait()


def _out_start(bufs, hbm_refs, off, n, sem):
    for b, h in zip(bufs, hbm_refs):
        pltpu.async_copy(b.at[pl.ds(0, n)], h.at[pl.ds(off, n)], sem)


def _out_wait(bufs, hbm_refs, off, n, sem):
    for b, h in zip(bufs, hbm_refs):
        pltpu.make_async_copy(b.at[pl.ds(0, n)], h.at[pl.ds(off, n)],
                              sem).wait()


def _stage_x(p_hbm, b_hbm, x_l, seta, setb, sema, semb, nring, tails):
    def srcs(c):
        off = c * SCH
        return (p_hbm.at[pl.ds(off, SCH)],
                p_hbm.at[pl.ds(NPAD + off, SCH)],
                b_hbm.at[pl.ds(off, SCH)])

    def start(c, bufs, sem):
        for h, b in zip(srcs(c), bufs):
            pltpu.async_copy(h, b.at[pl.ds(0, SCH)], sem)

    def wait(c, bufs, sem):
        for h, b in zip(srcs(c), bufs):
            pltpu.make_async_copy(h, b.at[pl.ds(0, SCH)], sem).wait()

    def accum(c, bufs):
        fa, fb, fc = bufs

        def vec(j, _):
            sl = pl.ds(j * 16, 16)
            x_l[pl.ds(c * SCH + j * 16, 16)] = fa[sl] + fb[sl] + fc[sl]
            return 0

        lax.fori_loop(0, SCH // 16, vec, 0)

    start(0, seta, sema)
    start(1, setb, semb)

    def body(k, _):
        c0 = 2 * k
        c1 = c0 + 1
        wait(c0, seta, sema)
        accum(c0, seta)

        @pl.when(c0 + 2 < nring)
        def _():
            start(c0 + 2, seta, sema)

        wait(c1, setb, semb)
        accum(c1, setb)

        @pl.when(c1 + 2 < nring)
        def _():
            start(c1 + 2, setb, semb)

        return 0

    lax.fori_loop(0, nring // 2, body, 0)
    fa, fb, fc = seta
    for off, sz in tails:
        pltpu.sync_copy(p_hbm.at[pl.ds(off, sz)], fa.at[pl.ds(0, sz)])
        pltpu.sync_copy(p_hbm.at[pl.ds(NPAD + off, sz)], fb.at[pl.ds(0, sz)])
        pltpu.sync_copy(b_hbm.at[pl.ds(off, sz)], fc.at[pl.ds(0, sz)])

        def tvec(j, _, off=off):
            sl = pl.ds(j * 16, 16)
            x_l[pl.ds(off + j * 16, 16)] = fa[sl] + fb[sl] + fc[sl]
            return 0

        lax.fori_loop(0, sz // 16, tvec, 0)


def _prep_body(src_hbm, dst_hbm, rew_hbm, mu_hbm, m_hbm, pw_hbm,
               mu_l, s0, s1, d0, d1, r0, r1, m0, m1, q0, q1,
               sia, sib, soa, sob):
    cid = lax.axis_index("c")
    sid = lax.axis_index("s")
    wid = cid * NSUB + sid
    pltpu.sync_copy(mu_hbm, mu_l)
    ebase = wid * EPT
    ins = (src_hbm, dst_hbm, rew_hbm)
    outs = (m_hbm, pw_hbm)

    def compute(sb, db, rb, mb, qb):
        def vec(i, _):
            sl = pl.ds(i * 16, 16)
            mu_j = plsc.load_gather(mu_l, [sb[sl]])
            mu_i = plsc.load_gather(mu_l, [db[sl]])
            inv = 1.0 / mu_i
            mb[sl] = jnp.exp(rb[sl] * inv)
            qb[sl] = mu_j * inv
            return 0

        lax.fori_loop(0, VPC, vec, 0)

    _in_start(ins, (s0, d0, r0), ebase, CH, sia)
    _in_start(ins, (s1, d1, r1), ebase + CH, CH, sib)

    def body(k, _):
        c0 = 2 * k
        c1 = c0 + 1
        o0 = ebase + c0 * CH
        o1 = ebase + c1 * CH
        _in_wait(ins, (s0, d0, r0), o0, CH, sia)

        @pl.when(c0 >= 2)
        def _():
            _out_wait((m0, q0), outs, o0 - 2 * CH, CH, soa)

        compute(s0, d0, r0, m0, q0)
        _out_start((m0, q0), outs, o0, CH, soa)

        @pl.when(c0 + 2 < NCH)
        def _():
            _in_start(ins, (s0, d0, r0), o0 + 2 * CH, CH, sia)

        _in_wait(ins, (s1, d1, r1), o1, CH, sib)

        @pl.when(c1 >= 2)
        def _():
            _out_wait((m1, q1), outs, o1 - 2 * CH, CH, sob)

        compute(s1, d1, r1, m1, q1)
        _out_start((m1, q1), outs, o1, CH, sob)

        @pl.when(c1 + 2 < NCH)
        def _():
            _in_start(ins, (s1, d1, r1), o1 + 2 * CH, CH, sib)

        return 0

    lax.fori_loop(0, NCH // 2, body, 0)
    _out_wait((m0, q0), outs, ebase + (NCH - 2) * CH, CH, soa)
    _out_wait((m1, q1), outs, ebase + (NCH - 1) * CH, CH, sob)


def _prep_sc(src, dst, rewards, mu_pad):
    return pl.kernel(
        _prep_body,
        out_type=(jax.ShapeDtypeStruct((E,), jnp.float32),
                  jax.ShapeDtypeStruct((E,), jnp.float32)),
        mesh=_mesh,
        compiler_params=_sc_params,
        scratch_types=[
            pltpu.VMEM((NPAD,), jnp.float32),
            pltpu.VMEM((CH,), jnp.int32),
            pltpu.VMEM((CH,), jnp.int32),
            pltpu.VMEM((CH,), jnp.int32),
            pltpu.VMEM((CH,), jnp.int32),
            pltpu.VMEM((CH,), jnp.float32),
            pltpu.VMEM((CH,), jnp.float32),
            pltpu.VMEM((CH,), jnp.float32),
            pltpu.VMEM((CH,), jnp.float32),
            pltpu.VMEM((CH,), jnp.float32),
            pltpu.VMEM((CH,), jnp.float32),
            pltpu.SemaphoreType.DMA,
            pltpu.SemaphoreType.DMA,
            pltpu.SemaphoreType.DMA,
            pltpu.SemaphoreType.DMA,
        ],
    )(src, dst, rewards, mu_pad)


def _iter_body(p_hbm, b_hbm, src_hbm, dst_hbm, m_hbm, pw_hbm, pout_hbm,
               x_l, s0, s1, d0, d1, m0, m1, q0, q1, g0, g1, e0, e1,
               acc, sia, sib, ssa, ssb):
    cid = lax.axis_index("c")
    sid = lax.axis_index("s")
    wid = cid * NSUB + sid
    def zvec(j, _):
        g0[pl.ds(j * 16, 16)] = jnp.zeros((16,), jnp.float32)
        return 0

    lax.fori_loop(0, SCH // 16, zvec, 0)
    for part in range(4):
        pltpu.sync_copy(g0.at[pl.ds(0, SCH)],
                        acc.at[pl.ds(sid * SLICE + part * SCH, SCH)])
    _stage_x(p_hbm, b_hbm, x_l, (m0, q0, g0), (m1, q1, g1), sia, sib,
             62, ((97216, 1568), (98784, 1216)))
    plsc.subcore_barrier()

    ebase = wid * EPT
    ins = (src_hbm, dst_hbm, m_hbm, pw_hbm)

    def compute(sb, db, mb, qb, gb, eb):
        def vec(i, _):
            sl = pl.ds(i * 16, 16)
            xs = plsc.load_gather(x_l, [sb[sl]])
            gb[sl] = mb[sl] * _vpow(xs, qb[sl])
            eb[sl] = db[sl]
            return 0

        lax.fori_loop(0, VPC, vec, 0)

    _in_start(ins, (s0, d0, m0, q0), ebase, CH, sia)
    _in_start(ins, (s1, d1, m1, q1), ebase + CH, CH, sib)

    def body(k, _):
        c0 = 2 * k
        c1 = c0 + 1
        o0 = ebase + c0 * CH
        o1 = ebase + c1 * CH
        _in_wait(ins, (s0, d0, m0, q0), o0, CH, sia)
        compute(s0, d0, m0, q0, g0, e0)

        @pl.when(c0 >= 2)
        def _():
            pltpu.make_async_copy(g0, acc.at[e0], ssa).wait()

        pltpu.async_copy(g0, acc.at[e0], ssa, add=True)

        @pl.when(c0 + 2 < NCH)
        def _():
            _in_start(ins, (s0, d0, m0, q0), o0 + 2 * CH, CH, sia)

        _in_wait(ins, (s1, d1, m1, q1), o1, CH, sib)
        compute(s1, d1, m1, q1, g1, e1)

        @pl.when(c1 >= 2)
        def _():
            pltpu.make_async_copy(g1, acc.at[e1], ssb).wait()

        pltpu.async_copy(g1, acc.at[e1], ssb, add=True)

        @pl.when(c1 + 2 < NCH)
        def _():
            _in_start(ins, (s1, d1, m1, q1), o1 + 2 * CH, CH, sib)

        return 0

    lax.fori_loop(0, NCH // 2, body, 0)
    pltpu.make_async_copy(g0, acc.at[e0], ssa).wait()
    pltpu.make_async_copy(g1, acc.at[e1], ssb).wait()
    plsc.subcore_barrier()
    for part in range(4):
        pltpu.sync_copy(acc.at[pl.ds(sid * SLICE + part * SCH, SCH)],
                        g0.at[pl.ds(0, SCH)])
        pltpu.sync_copy(
            g0.at[pl.ds(0, SCH)],
            pout_hbm.at[pl.ds(cid * NPAD + sid * SLICE + part * SCH, SCH)])


def _iter_sc(p_prev, b_pad, src, dst, M, pw):
    return pl.kernel(
        _iter_body,
        out_type=jax.ShapeDtypeStruct((2 * NPAD,), jnp.float32),
        mesh=_mesh,
        compiler_params=_sc_params,
        scratch_types=[
            pltpu.VMEM((N,), jnp.float32),
            pltpu.VMEM((CH,), jnp.int32),
            pltpu.VMEM((CH,), jnp.int32),
            pltpu.VMEM((CH,), jnp.int32),
            pltpu.VMEM((CH,), jnp.int32),
            pltpu.VMEM((CH,), jnp.float32),
            pltpu.VMEM((CH,), jnp.float32),
            pltpu.VMEM((CH,), jnp.float32),
            pltpu.VMEM((CH,), jnp.float32),
            pltpu.VMEM((CH,), jnp.float32),
            pltpu.VMEM((CH,), jnp.float32),
            pltpu.VMEM((CH,), jnp.int32),
            pltpu.VMEM((CH,), jnp.int32),
            pltpu.VMEM_SHARED((NPAD,), jnp.float32),
            pltpu.SemaphoreType.DMA,
            pltpu.SemaphoreType.DMA,
            pltpu.SemaphoreType.DMA,
            pltpu.SemaphoreType.DMA,
        ],
    )(p_prev, b_pad, src, dst, M, pw)


def _epi_body(p_hbm, b_hbm, src_hbm, dst_hbm, m_hbm, pw_hbm,
              val_hbm, ep_hbm,
              x_l, s0, s1, d0, d1, m0, m1, q0, q1, g0, g1, vbuf,
              sia, sib, soa, sob):
    cid = lax.axis_index("c")
    sid = lax.axis_index("s")
    wid = cid * NSUB + sid
    _stage_x(p_hbm, b_hbm, x_l, (m0, q0, g0), (m1, q1, g1), sia, sib,
             NSCH, ())

    vbase = wid * VSLICE

    def vvec(j, _):
        xv = x_l[pl.ds(vbase + j * 16, 16)]
        lv = _vlog(xv)
        lv = jnp.where(xv == 0.0,
                       jnp.full((16,), -jnp.inf, jnp.float32), lv)
        vbuf[pl.ds(j * 16, 16)] = lv
        return 0

    lax.fori_loop(0, VSLICE // 16, vvec, 0)
    pltpu.sync_copy(vbuf, val_hbm.at[pl.ds(vbase, VSLICE)])

    ebase = wid * EPT
    ins = (src_hbm, dst_hbm, m_hbm, pw_hbm)

    def compute(sb, db, mb, qb, gb):
        def vec(i, _):
            sl = pl.ds(i * 16, 16)
            xs = plsc.load_gather(x_l, [sb[sl]])
            xd = plsc.load_gather(x_l, [db[sl]])
            gb[sl] = mb[sl] * _vpow(xs, qb[sl]) / xd
            return 0

        lax.fori_loop(0, VPC, vec, 0)

    _in_start(ins, (s0, d0, m0, q0), ebase, CH, sia)
    _in_start(ins, (s1, d1, m1, q1), ebase + CH, CH, sib)

    def body(k, _):
        c0 = 2 * k
        c1 = c0 + 1
        o0 = ebase + c0 * CH
        o1 = ebase + c1 * CH
        _in_wait(ins, (s0, d0, m0, q0), o0, CH, sia)

        @pl.when(c0 >= 2)
        def _():
            _out_wait((g0,), (ep_hbm,), o0 - 2 * CH, CH, soa)

        compute(s0, d0, m0, q0, g0)
        _out_start((g0,), (ep_hbm,), o0, CH, soa)

        @pl.when(c0 + 2 < NCH)
        def _():
            _in_start(ins, (s0, d0, m0, q0), o0 + 2 * CH, CH, sia)

        _in_wait(ins, (s1, d1, m1, q1), o1, CH, sib)

        @pl.when(c1 >= 2)
        def _():
            _out_wait((g1,), (ep_hbm,), o1 - 2 * CH, CH, sob)

        compute(s1, d1, m1, q1, g1)
        _out_start((g1,), (ep_hbm,), o1, CH, sob)

        @pl.when(c1 + 2 < NCH)
        def _():
            _in_start(ins, (s1, d1, m1, q1), o1 + 2 * CH, CH, sib)

        return 0

    lax.fori_loop(0, NCH // 2, body, 0)
    _out_wait((g0,), (ep_hbm,), ebase + (NCH - 2) * CH, CH, soa)
    _out_wait((g1,), (ep_hbm,), ebase + (NCH - 1) * CH, CH, sob)


def _epi_sc(p_last, b_pad, src, dst, M, pw):
    return pl.kernel(
        _epi_body,
        out_type=(jax.ShapeDtypeStruct((NPAD,), jnp.float32),
                  jax.ShapeDtypeStruct((E,), jnp.float32)),
        mesh=_mesh,
        compiler_params=_sc_params,
        scratch_types=[
            pltpu.VMEM((NPAD,), jnp.float32),
            pltpu.VMEM((CH,), jnp.int32),
            pltpu.VMEM((CH,), jnp.int32),
            pltpu.VMEM((CH,), jnp.int32),
            pltpu.VMEM((CH,), jnp.int32),
            pltpu.VMEM((CH,), jnp.float32),
            pltpu.VMEM((CH,), jnp.float32),
            pltpu.VMEM((CH,), jnp.float32),
            pltpu.VMEM((CH,), jnp.float32),
            pltpu.VMEM((CH,), jnp.float32),
            pltpu.VMEM((CH,), jnp.float32),
            pltpu.VMEM((VSLICE,), jnp.float32),
            pltpu.SemaphoreType.DMA,
            pltpu.SemaphoreType.DMA,
            pltpu.SemaphoreType.DMA,
            pltpu.SemaphoreType.DMA,
        ],
    )(p_last, b_pad, src, dst, M, pw)


def kernel(edge_index, edge_feats, node_scales, sink_node_mask, W, b0):
    src = edge_index[0].astype(jnp.int32)
    dst = edge_index[1].astype(jnp.int32)
    pad = NPAD - N
    mu_pad = jnp.concatenate(
        [node_scales, jnp.ones((pad,), jnp.float32)])
    b_pad = jnp.concatenate(
        [sink_node_mask, jnp.zeros((pad,), jnp.float32)])

    rewards = _rewards_tc(edge_feats, W, b0)
    M, pw = _prep_sc(src, dst, rewards, mu_pad)

    p0 = jnp.zeros((2 * NPAD,), jnp.float32)
    p_last = lax.fori_loop(
        0, N_ITERS,
        lambda i, p: _iter_sc(p, b_pad, src, dst, M, pw),
        p0)

    values_pad, edge_probs = _epi_sc(p_last, b_pad, src, dst, M, pw)
    return rewards, values_pad[:N], edge_probs


# speedup vs baseline: 89.3103x; 1.3077x over previous
import jax
import jax.numpy as jnp
from jax import lax
from jax.experimental import pallas as pl
from jax.experimental.pallas import tpu as pltpu
from jax.experimental.pallas import tpu_sc as plsc

N = 100000
E = 3200000
N_ITERS = 12

NCORES = 2
NSUB = 16
NTILES = NCORES * NSUB
LN2 = 0.6931471805599453

NPAD = 100352
SLICE = NPAD // NSUB
VSLICE = NPAD // NTILES
SCH = 1568
NSCH = NPAD // SCH
EPT = E // NTILES
CH = 2000
NCH = EPT // CH
VPC = CH // 16

_mesh = plsc.VectorSubcoreMesh(core_axis_name="c", subcore_axis_name="s")
_sc_params = pltpu.CompilerParams(needs_layout_passes=False)


def _vlog(xv):
    xv = jnp.maximum(xv, jnp.float32(1e-12))
    bits = plsc.bitcast(xv, jnp.int32)
    e = (bits >> 23) - 127
    m = plsc.bitcast((bits & 0x7FFFFF) | 0x3F800000, jnp.float32)
    big = m > jnp.float32(1.4142135)
    m = jnp.where(big, m * jnp.float32(0.5), m)
    ef = (e + big.astype(jnp.int32)).astype(jnp.float32)
    s = (m - 1.0) / (m + 1.0)
    s2 = s * s
    p = jnp.float32(2.0 / 9.0)
    p = p * s2 + jnp.float32(2.0 / 7.0)
    p = p * s2 + jnp.float32(2.0 / 5.0)
    p = p * s2 + jnp.float32(2.0 / 3.0)
    p = p * s2 + jnp.float32(2.0)
    return ef * jnp.float32(LN2) + p * s


def _vpow(xv, pv):
    return jnp.exp(pv * _vlog(xv))


def _rewards_body(ef_ref, wb_ref, b0_ref, out_ref):
    enc = jnp.dot(ef_ref[...], wb_ref[...],
                  preferred_element_type=jnp.float32) + b0_ref[0]
    out_ref[...] = -jnp.logaddexp(enc, 0.0)


def _rewards_tc(ef2, wb, b0):
    rows = E // 8
    blk = 8000
    out = pl.pallas_call(
        _rewards_body,
        grid=(rows // blk,),
        in_specs=[
            pl.BlockSpec((blk, 128), lambda i: (i, 0)),
            pl.BlockSpec((128, 8), lambda i: (0, 0)),
            pl.BlockSpec(memory_space=pltpu.SMEM),
        ],
        out_specs=pl.BlockSpec((blk, 8), lambda i: (i, 0)),
        out_shape=jax.ShapeDtypeStruct((rows, 8), jnp.float32),
        compiler_params=pltpu.CompilerParams(
            dimension_semantics=("arbitrary",)),
        name="tc_rewards",
    )(ef2, wb, b0)
    return out.reshape(E)


def _split_body(ei_ref, src_ref, dst_ref):
    src_ref[...] = ei_ref[0, :]
    dst_ref[...] = ei_ref[1, :]


def _split_tc(edge_index):
    blk = 128000
    return pl.pallas_call(
        _split_body,
        grid=(E // blk,),
        in_specs=[pl.BlockSpec((2, blk), lambda i: (0, i))],
        out_specs=(pl.BlockSpec((blk,), lambda i: (i,)),
                   pl.BlockSpec((blk,), lambda i: (i,))),
        out_shape=(jax.ShapeDtypeStruct((E,), jnp.int32),
                   jax.ShapeDtypeStruct((E,), jnp.int32)),
        compiler_params=pltpu.CompilerParams(
            dimension_semantics=("arbitrary",)),
        name="tc_split",
    )(edge_index)


def _in_start(hbm_refs, bufs, off, n, sem):
    for h, b in zip(hbm_refs, bufs):
        pltpu.async_copy(h.at[pl.ds(off, n)], b.at[pl.ds(0, n)], sem)


def _in_wait(hbm_refs, bufs, off, n, sem):
    for h, b in zip(hbm_refs, bufs):
        pltpu.make_async_copy(h.at[pl.ds(off, n)], b.at[pl.ds(0, n)],
                              sem).wait()


def _out_start(bufs, hbm_refs, off, n, sem):
    for b, h in zip(bufs, hbm_refs):
        pltpu.async_copy(b.at[pl.ds(0, n)], h.at[pl.ds(off, n)], sem)


def _out_wait(bufs, hbm_refs, off, n, sem):
    for b, h in zip(bufs, hbm_refs):
        pltpu.make_async_copy(b.at[pl.ds(0, n)], h.at[pl.ds(off, n)],
                              sem).wait()


def _stage_x(p_hbm, b_hbm, x_l, seta, setb, sema, semb, nring, tails):
    def srcs(c):
        off = c * SCH
        return (p_hbm.at[pl.ds(off, SCH)],
                p_hbm.at[pl.ds(NPAD + off, SCH)],
                b_hbm.at[pl.ds(off, SCH)])

    def start(c, bufs, sem):
        for h, b in zip(srcs(c), bufs):
            pltpu.async_copy(h, b.at[pl.ds(0, SCH)], sem)

    def wait(c, bufs, sem):
        for h, b in zip(srcs(c), bufs):
            pltpu.make_async_copy(h, b.at[pl.ds(0, SCH)], sem).wait()

    def accum(c, bufs):
        fa, fb, fc = bufs

        def vec(j, _):
            for u in range(7):
                sl = pl.ds(j * 112 + u * 16, 16)
                x_l[pl.ds(c * SCH + j * 112 + u * 16, 16)] = (
                    fa[sl] + fb[sl] + fc[sl])
            return 0

        lax.fori_loop(0, SCH // 112, vec, 0)

    start(0, seta, sema)
    start(1, setb, semb)

    def body(k, _):
        c0 = 2 * k
        c1 = c0 + 1
        wait(c0, seta, sema)
        accum(c0, seta)

        @pl.when(c0 + 2 < nring)
        def _():
            start(c0 + 2, seta, sema)

        wait(c1, setb, semb)
        accum(c1, setb)

        @pl.when(c1 + 2 < nring)
        def _():
            start(c1 + 2, setb, semb)

        return 0

    lax.fori_loop(0, nring // 2, body, 0)
    fa, fb, fc = seta
    for off, sz in tails:
        pltpu.sync_copy(p_hbm.at[pl.ds(off, sz)], fa.at[pl.ds(0, sz)])
        pltpu.sync_copy(p_hbm.at[pl.ds(NPAD + off, sz)], fb.at[pl.ds(0, sz)])
        pltpu.sync_copy(b_hbm.at[pl.ds(off, sz)], fc.at[pl.ds(0, sz)])

        def tvec(j, _, off=off):
            sl = pl.ds(j * 16, 16)
            x_l[pl.ds(off + j * 16, 16)] = fa[sl] + fb[sl] + fc[sl]
            return 0

        lax.fori_loop(0, sz // 16, tvec, 0)


def _prep_body(src_hbm, dst_hbm, rew_hbm, mu_hbm, m_hbm, pw_hbm,
               mu_l, s0, s1, d0, d1, r0, r1, m0, m1, q0, q1,
               sia, sib, soa, sob):
    cid = lax.axis_index("c")
    sid = lax.axis_index("s")
    wid = cid * NSUB + sid
    pltpu.sync_copy(mu_hbm, mu_l)
    ebase = wid * EPT
    ins = (src_hbm, dst_hbm, rew_hbm)
    outs = (m_hbm, pw_hbm)

    def compute(sb, db, rb, mb, qb):
        def vec(i, _):
            for u in range(5):
                sl = pl.ds(i * 80 + u * 16, 16)
                mu_j = plsc.load_gather(mu_l, [sb[sl]])
                mu_i = plsc.load_gather(mu_l, [db[sl]])
                inv = 1.0 / mu_i
                mb[sl] = jnp.exp(rb[sl] * inv)
                qb[sl] = mu_j * inv
            return 0

        lax.fori_loop(0, VPC // 5, vec, 0)

    _in_start(ins, (s0, d0, r0), ebase, CH, sia)
    _in_start(ins, (s1, d1, r1), ebase + CH, CH, sib)

    def body(k, _):
        c0 = 2 * k
        c1 = c0 + 1
        o0 = ebase + c0 * CH
        o1 = ebase + c1 * CH
        _in_wait(ins, (s0, d0, r0), o0, CH, sia)

        @pl.when(c0 >= 2)
        def _():
            _out_wait((m0, q0), outs, o0 - 2 * CH, CH, soa)

        compute(s0, d0, r0, m0, q0)
        _out_start((m0, q0), outs, o0, CH, soa)

        @pl.when(c0 + 2 < NCH)
        def _():
            _in_start(ins, (s0, d0, r0), o0 + 2 * CH, CH, sia)

        _in_wait(ins, (s1, d1, r1), o1, CH, sib)

        @pl.when(c1 >= 2)
        def _():
            _out_wait((m1, q1), outs, o1 - 2 * CH, CH, sob)

        compute(s1, d1, r1, m1, q1)
        _out_start((m1, q1), outs, o1, CH, sob)

        @pl.when(c1 + 2 < NCH)
        def _():
            _in_start(ins, (s1, d1, r1), o1 + 2 * CH, CH, sib)

        return 0

    lax.fori_loop(0, NCH // 2, body, 0)
    _out_wait((m0, q0), outs, ebase + (NCH - 2) * CH, CH, soa)
    _out_wait((m1, q1), outs, ebase + (NCH - 1) * CH, CH, sob)


def _prep_sc(src, dst, rewards, mu_pad):
    return pl.kernel(
        _prep_body,
        out_type=(jax.ShapeDtypeStruct((E,), jnp.float32),
                  jax.ShapeDtypeStruct((E,), jnp.float32)),
        mesh=_mesh,
        compiler_params=_sc_params,
        name="sc_prep",
        scratch_types=[
            pltpu.VMEM((NPAD,), jnp.float32),
            pltpu.VMEM((CH,), jnp.int32),
            pltpu.VMEM((CH,), jnp.int32),
            pltpu.VMEM((CH,), jnp.int32),
            pltpu.VMEM((CH,), jnp.int32),
            pltpu.VMEM((CH,), jnp.float32),
            pltpu.VMEM((CH,), jnp.float32),
            pltpu.VMEM((CH,), jnp.float32),
            pltpu.VMEM((CH,), jnp.float32),
            pltpu.VMEM((CH,), jnp.float32),
            pltpu.VMEM((CH,), jnp.float32),
            pltpu.SemaphoreType.DMA,
            pltpu.SemaphoreType.DMA,
            pltpu.SemaphoreType.DMA,
            pltpu.SemaphoreType.DMA,
        ],
    )(src, dst, rewards, mu_pad)


def _iter_body(p_hbm, b_hbm, src_hbm, dst_hbm, m_hbm, pw_hbm, pout_hbm,
               x_l, s0, s1, d0, d1, m0, m1, q0, q1, g0, g1, e0, e1,
               acc, sia, sib, ssa, ssb):
    cid = lax.axis_index("c")
    sid = lax.axis_index("s")
    wid = cid * NSUB + sid
    def zvec(j, _):
        g0[pl.ds(j * 16, 16)] = jnp.zeros((16,), jnp.float32)
        return 0

    lax.fori_loop(0, SCH // 16, zvec, 0)
    for part in range(4):
        pltpu.sync_copy(g0.at[pl.ds(0, SCH)],
                        acc.at[pl.ds(sid * SLICE + part * SCH, SCH)])
    _stage_x(p_hbm, b_hbm, x_l, (m0, q0, g0), (m1, q1, g1), sia, sib,
             62, ((97216, 1568), (98784, 1216)))
    plsc.subcore_barrier()

    ebase = wid * EPT
    ins = (src_hbm, dst_hbm, m_hbm, pw_hbm)

    def compute(sb, db, mb, qb, gb, eb):
        def vec(i, _):
            for u in range(5):
                sl = pl.ds(i * 80 + u * 16, 16)
                xs = plsc.load_gather(x_l, [sb[sl]])
                gb[sl] = mb[sl] * _vpow(xs, qb[sl])
                eb[sl] = db[sl]
            return 0

        lax.fori_loop(0, VPC // 5, vec, 0)

    _in_start(ins, (s0, d0, m0, q0), ebase, CH, sia)
    _in_start(ins, (s1, d1, m1, q1), ebase + CH, CH, sib)

    def body(k, _):
        c0 = 2 * k
        c1 = c0 + 1
        o0 = ebase + c0 * CH
        o1 = ebase + c1 * CH
        _in_wait(ins, (s0, d0, m0, q0), o0, CH, sia)
        compute(s0, d0, m0, q0, g0, e0)

        @pl.when(c0 >= 2)
        def _():
            pltpu.make_async_copy(g0, acc.at[e0], ssa).wait()

        pltpu.async_copy(g0, acc.at[e0], ssa, add=True)

        @pl.when(c0 + 2 < NCH)
        def _():
            _in_start(ins, (s0, d0, m0, q0), o0 + 2 * CH, CH, sia)

        _in_wait(ins, (s1, d1, m1, q1), o1, CH, sib)
        compute(s1, d1, m1, q1, g1, e1)

        @pl.when(c1 >= 2)
        def _():
            pltpu.make_async_copy(g1, acc.at[e1], ssb).wait()

        pltpu.async_copy(g1, acc.at[e1], ssb, add=True)

        @pl.when(c1 + 2 < NCH)
        def _():
            _in_start(ins, (s1, d1, m1, q1), o1 + 2 * CH, CH, sib)

        return 0

    lax.fori_loop(0, NCH // 2, body, 0)
    pltpu.make_async_copy(g0, acc.at[e0], ssa).wait()
    pltpu.make_async_copy(g1, acc.at[e1], ssb).wait()
    plsc.subcore_barrier()
    for part in range(4):
        pltpu.sync_copy(acc.at[pl.ds(sid * SLICE + part * SCH, SCH)],
                        g0.at[pl.ds(0, SCH)])
        pltpu.sync_copy(
            g0.at[pl.ds(0, SCH)],
            pout_hbm.at[pl.ds(cid * NPAD + sid * SLICE + part * SCH, SCH)])


def _iter_sc(p_prev, b_pad, src, dst, M, pw):
    return pl.kernel(
        _iter_body,
        out_type=jax.ShapeDtypeStruct((2 * NPAD,), jnp.float32),
        mesh=_mesh,
        compiler_params=_sc_params,
        name="sc_iter",
        scratch_types=[
            pltpu.VMEM((N,), jnp.float32),
            pltpu.VMEM((CH,), jnp.int32),
            pltpu.VMEM((CH,), jnp.int32),
            pltpu.VMEM((CH,), jnp.int32),
            pltpu.VMEM((CH,), jnp.int32),
            pltpu.VMEM((CH,), jnp.float32),
            pltpu.VMEM((CH,), jnp.float32),
            pltpu.VMEM((CH,), jnp.float32),
            pltpu.VMEM((CH,), jnp.float32),
            pltpu.VMEM((CH,), jnp.float32),
            pltpu.VMEM((CH,), jnp.float32),
            pltpu.VMEM((CH,), jnp.int32),
            pltpu.VMEM((CH,), jnp.int32),
            pltpu.VMEM_SHARED((NPAD,), jnp.float32),
            pltpu.SemaphoreType.DMA,
            pltpu.SemaphoreType.DMA,
            pltpu.SemaphoreType.DMA,
            pltpu.SemaphoreType.DMA,
        ],
    )(p_prev, b_pad, src, dst, M, pw)


def _epi_body(p_hbm, b_hbm, src_hbm, dst_hbm, m_hbm, pw_hbm,
              val_hbm, ep_hbm,
              x_l, s0, s1, d0, d1, m0, m1, q0, q1, g0, g1, vbuf,
              sia, sib, soa, sob):
    cid = lax.axis_index("c")
    sid = lax.axis_index("s")
    wid = cid * NSUB + sid
    _stage_x(p_hbm, b_hbm, x_l, (m0, q0, g0), (m1, q1, g1), sia, sib,
             NSCH, ())

    vbase = wid * VSLICE

    def vvec(j, _):
        xv = x_l[pl.ds(vbase + j * 16, 16)]
        lv = _vlog(xv)
        lv = jnp.where(xv == 0.0,
                       jnp.full((16,), -jnp.inf, jnp.float32), lv)
        vbuf[pl.ds(j * 16, 16)] = lv
        return 0

    lax.fori_loop(0, VSLICE // 16, vvec, 0)
    pltpu.sync_copy(vbuf, val_hbm.at[pl.ds(vbase, VSLICE)])

    ebase = wid * EPT
    ins = (src_hbm, dst_hbm, m_hbm, pw_hbm)

    def compute(sb, db, mb, qb, gb):
        def vec(i, _):
            for u in range(5):
                sl = pl.ds(i * 80 + u * 16, 16)
                xs = plsc.load_gather(x_l, [sb[sl]])
                xd = plsc.load_gather(x_l, [db[sl]])
                gb[sl] = mb[sl] * _vpow(xs, qb[sl]) / xd
            return 0

        lax.fori_loop(0, VPC // 5, vec, 0)

    _in_start(ins, (s0, d0, m0, q0), ebase, CH, sia)
    _in_start(ins, (s1, d1, m1, q1), ebase + CH, CH, sib)

    def body(k, _):
        c0 = 2 * k
        c1 = c0 + 1
        o0 = ebase + c0 * CH
        o1 = ebase + c1 * CH
        _in_wait(ins, (s0, d0, m0, q0), o0, CH, sia)

        @pl.when(c0 >= 2)
        def _():
            _out_wait((g0,), (ep_hbm,), o0 - 2 * CH, CH, soa)

        compute(s0, d0, m0, q0, g0)
        _out_start((g0,), (ep_hbm,), o0, CH, soa)

        @pl.when(c0 + 2 < NCH)
        def _():
            _in_start(ins, (s0, d0, m0, q0), o0 + 2 * CH, CH, sia)

        _in_wait(ins, (s1, d1, m1, q1), o1, CH, sib)

        @pl.when(c1 >= 2)
        def _():
            _out_wait((g1,), (ep_hbm,), o1 - 2 * CH, CH, sob)

        compute(s1, d1, m1, q1, g1)
        _out_start((g1,), (ep_hbm,), o1, CH, sob)

        @pl.when(c1 + 2 < NCH)
        def _():
            _in_start(ins, (s1, d1, m1, q1), o1 + 2 * CH, CH, sib)

        return 0

    lax.fori_loop(0, NCH // 2, body, 0)
    _out_wait((g0,), (ep_hbm,), ebase + (NCH - 2) * CH, CH, soa)
    _out_wait((g1,), (ep_hbm,), ebase + (NCH - 1) * CH, CH, sob)


def _epi_sc(p_last, b_pad, src, dst, M, pw):
    return pl.kernel(
        _epi_body,
        out_type=(jax.ShapeDtypeStruct((NPAD,), jnp.float32),
                  jax.ShapeDtypeStruct((E,), jnp.float32)),
        mesh=_mesh,
        compiler_params=_sc_params,
        name="sc_epi",
        scratch_types=[
            pltpu.VMEM((NPAD,), jnp.float32),
            pltpu.VMEM((CH,), jnp.int32),
            pltpu.VMEM((CH,), jnp.int32),
            pltpu.VMEM((CH,), jnp.int32),
            pltpu.VMEM((CH,), jnp.int32),
            pltpu.VMEM((CH,), jnp.float32),
            pltpu.VMEM((CH,), jnp.float32),
            pltpu.VMEM((CH,), jnp.float32),
            pltpu.VMEM((CH,), jnp.float32),
            pltpu.VMEM((CH,), jnp.float32),
            pltpu.VMEM((CH,), jnp.float32),
            pltpu.VMEM((VSLICE,), jnp.float32),
            pltpu.SemaphoreType.DMA,
            pltpu.SemaphoreType.DMA,
            pltpu.SemaphoreType.DMA,
            pltpu.SemaphoreType.DMA,
        ],
    )(p_last, b_pad, src, dst, M, pw)


def kernel(edge_index, edge_feats, node_scales, sink_node_mask, W, b0):
    src, dst = _split_tc(edge_index.astype(jnp.int32))
    pad = NPAD - N
    mu_pad = jnp.concatenate(
        [node_scales, jnp.ones((pad,), jnp.float32)])
    b_pad = jnp.concatenate(
        [sink_node_mask, jnp.zeros((pad,), jnp.float32)])

    ef2 = edge_feats.reshape(E // 8, 128)
    wb = jnp.kron(jnp.eye(8, dtype=jnp.float32), W)
    rewards = _rewards_tc(ef2, wb, b0)
    M, pw = _prep_sc(src, dst, rewards, mu_pad)

    p0 = jnp.zeros((2 * NPAD,), jnp.float32)
    p_last = lax.fori_loop(
        0, N_ITERS,
        lambda i, p: _iter_sc(p, b_pad, src, dst, M, pw),
        p0)

    values_pad, edge_probs = _epi_sc(p_last, b_pad, src, dst, M, pw)
    return rewards, values_pad[:N], edge_probs


# speedup vs baseline: 176.0789x; 1.9715x over previous
import jax
import jax.numpy as jnp
from jax import lax
from jax.experimental import pallas as pl
from jax.experimental.pallas import tpu as pltpu
from jax.experimental.pallas import tpu_sc as plsc

N = 100000
E = 3200000
N_ITERS = 12

NCORES = 2
NSUB = 16
NTILES = NCORES * NSUB
LN2 = 0.6931471805599453

NPAD = 100352
SLICE = NPAD // NSUB
VSLICE = NPAD // NTILES
SCH = 1568
NSCH = NPAD // SCH
EPT = E // NTILES
CH = 2000
NCH = EPT // CH
VPC = CH // 16

_mesh = plsc.VectorSubcoreMesh(core_axis_name="c", subcore_axis_name="s")
_sc_params = pltpu.CompilerParams(needs_layout_passes=False)


def _vlog(xv):
    xv = jnp.maximum(xv, jnp.float32(1e-12))
    bits = plsc.bitcast(xv, jnp.int32)
    e = (bits >> 23) - 127
    m = plsc.bitcast((bits & 0x7FFFFF) | 0x3F800000, jnp.float32)
    big = m > jnp.float32(1.4142135)
    m = jnp.where(big, m * jnp.float32(0.5), m)
    ef = (e + big.astype(jnp.int32)).astype(jnp.float32)
    s = (m - 1.0) / (m + 1.0)
    s2 = s * s
    p = jnp.float32(2.0 / 9.0)
    p = p * s2 + jnp.float32(2.0 / 7.0)
    p = p * s2 + jnp.float32(2.0 / 5.0)
    p = p * s2 + jnp.float32(2.0 / 3.0)
    p = p * s2 + jnp.float32(2.0)
    return ef * jnp.float32(LN2) + p * s


def _vpow(xv, pv):
    return jnp.exp(pv * _vlog(xv))


def _rewards_body(ef_ref, wb_ref, b0_ref, out_ref):
    enc = jnp.dot(ef_ref[...], wb_ref[...],
                  preferred_element_type=jnp.float32) + b0_ref[0]
    out_ref[...] = -jnp.logaddexp(enc, 0.0)


def _rewards_tc(ef2, wb, b0):
    rows = E // 8
    blk = 8000
    out = pl.pallas_call(
        _rewards_body,
        grid=(rows // blk,),
        in_specs=[
            pl.BlockSpec((blk, 128), lambda i: (i, 0)),
            pl.BlockSpec((128, 8), lambda i: (0, 0)),
            pl.BlockSpec(memory_space=pltpu.SMEM),
        ],
        out_specs=pl.BlockSpec((blk, 8), lambda i: (i, 0)),
        out_shape=jax.ShapeDtypeStruct((rows, 8), jnp.float32),
        compiler_params=pltpu.CompilerParams(
            dimension_semantics=("arbitrary",)),
        name="tc_rewards",
    )(ef2, wb, b0)
    return out.reshape(E)


def _split_body(ei_ref, src_ref, dst_ref):
    src_ref[...] = ei_ref[0, :]
    dst_ref[...] = ei_ref[1, :]


def _split_tc(edge_index):
    blk = 128000
    return pl.pallas_call(
        _split_body,
        grid=(E // blk,),
        in_specs=[pl.BlockSpec((2, blk), lambda i: (0, i))],
        out_specs=(pl.BlockSpec((blk,), lambda i: (i,)),
                   pl.BlockSpec((blk,), lambda i: (i,))),
        out_shape=(jax.ShapeDtypeStruct((E,), jnp.int32),
                   jax.ShapeDtypeStruct((E,), jnp.int32)),
        compiler_params=pltpu.CompilerParams(
            dimension_semantics=("arbitrary",)),
        name="tc_split",
    )(edge_index)


def _in_start(hbm_refs, bufs, off, n, sem):
    for h, b in zip(hbm_refs, bufs):
        pltpu.async_copy(h.at[pl.ds(off, n)], b.at[pl.ds(0, n)], sem)


def _in_wait(hbm_refs, bufs, off, n, sem):
    for h, b in zip(hbm_refs, bufs):
        pltpu.make_async_copy(h.at[pl.ds(off, n)], b.at[pl.ds(0, n)],
                              sem).wait()


def _out_start(bufs, hbm_refs, off, n, sem):
    for b, h in zip(bufs, hbm_refs):
        pltpu.async_copy(b.at[pl.ds(0, n)], h.at[pl.ds(off, n)], sem)


def _out_wait(bufs, hbm_refs, off, n, sem):
    for b, h in zip(bufs, hbm_refs):
        pltpu.make_async_copy(b.at[pl.ds(0, n)], h.at[pl.ds(off, n)],
                              sem).wait()


def _stage_x(p_hbm, b_hbm, x_l, seta, setb, sema, semb, nring, tails):
    def srcs(c):
        off = c * SCH
        return (p_hbm.at[pl.ds(off, SCH)],
                p_hbm.at[pl.ds(NPAD + off, SCH)],
                b_hbm.at[pl.ds(off, SCH)])

    def start(c, bufs, sem):
        for h, b in zip(srcs(c), bufs):
            pltpu.async_copy(h, b.at[pl.ds(0, SCH)], sem)

    def wait(c, bufs, sem):
        for h, b in zip(srcs(c), bufs):
            pltpu.make_async_copy(h, b.at[pl.ds(0, SCH)], sem).wait()

    def accum(c, bufs):
        fa, fb, fc = bufs

        @plsc.parallel_loop(0, SCH // 16, 1, unroll=7)
        def vec(j):
            sl = pl.ds(j * 16, 16)
            x_l[pl.ds(c * SCH + j * 16, 16)] = fa[sl] + fb[sl] + fc[sl]

    start(0, seta, sema)
    start(1, setb, semb)

    def body(k, _):
        c0 = 2 * k
        c1 = c0 + 1
        wait(c0, seta, sema)
        accum(c0, seta)

        @pl.when(c0 + 2 < nring)
        def _():
            start(c0 + 2, seta, sema)

        wait(c1, setb, semb)
        accum(c1, setb)

        @pl.when(c1 + 2 < nring)
        def _():
            start(c1 + 2, setb, semb)

        return 0

    lax.fori_loop(0, nring // 2, body, 0)
    fa, fb, fc = seta
    for off, sz in tails:
        pltpu.sync_copy(p_hbm.at[pl.ds(off, sz)], fa.at[pl.ds(0, sz)])
        pltpu.sync_copy(p_hbm.at[pl.ds(NPAD + off, sz)], fb.at[pl.ds(0, sz)])
        pltpu.sync_copy(b_hbm.at[pl.ds(off, sz)], fc.at[pl.ds(0, sz)])

        def tvec(j, _, off=off):
            sl = pl.ds(j * 16, 16)
            x_l[pl.ds(off + j * 16, 16)] = fa[sl] + fb[sl] + fc[sl]
            return 0

        lax.fori_loop(0, sz // 16, tvec, 0)


def _prep_body(src_hbm, dst_hbm, rew_hbm, mu_hbm, m_hbm, pw_hbm,
               mu_l, s0, s1, d0, d1, r0, r1, m0, m1, q0, q1,
               sia, sib, soa, sob):
    cid = lax.axis_index("c")
    sid = lax.axis_index("s")
    wid = cid * NSUB + sid
    pltpu.sync_copy(mu_hbm, mu_l)
    ebase = wid * EPT
    ins = (src_hbm, dst_hbm, rew_hbm)
    outs = (m_hbm, pw_hbm)

    def compute(sb, db, rb, mb, qb):
        @plsc.parallel_loop(0, VPC, 1, unroll=5)
        def vec(i):
            sl = pl.ds(i * 16, 16)
            mu_j = plsc.load_gather(mu_l, [sb[sl]])
            mu_i = plsc.load_gather(mu_l, [db[sl]])
            inv = 1.0 / mu_i
            mb[sl] = jnp.exp(rb[sl] * inv)
            qb[sl] = mu_j * inv

    _in_start(ins, (s0, d0, r0), ebase, CH, sia)
    _in_start(ins, (s1, d1, r1), ebase + CH, CH, sib)

    def body(k, _):
        c0 = 2 * k
        c1 = c0 + 1
        o0 = ebase + c0 * CH
        o1 = ebase + c1 * CH
        _in_wait(ins, (s0, d0, r0), o0, CH, sia)

        @pl.when(c0 >= 2)
        def _():
            _out_wait((m0, q0), outs, o0 - 2 * CH, CH, soa)

        compute(s0, d0, r0, m0, q0)
        _out_start((m0, q0), outs, o0, CH, soa)

        @pl.when(c0 + 2 < NCH)
        def _():
            _in_start(ins, (s0, d0, r0), o0 + 2 * CH, CH, sia)

        _in_wait(ins, (s1, d1, r1), o1, CH, sib)

        @pl.when(c1 >= 2)
        def _():
            _out_wait((m1, q1), outs, o1 - 2 * CH, CH, sob)

        compute(s1, d1, r1, m1, q1)
        _out_start((m1, q1), outs, o1, CH, sob)

        @pl.when(c1 + 2 < NCH)
        def _():
            _in_start(ins, (s1, d1, r1), o1 + 2 * CH, CH, sib)

        return 0

    lax.fori_loop(0, NCH // 2, body, 0)
    _out_wait((m0, q0), outs, ebase + (NCH - 2) * CH, CH, soa)
    _out_wait((m1, q1), outs, ebase + (NCH - 1) * CH, CH, sob)


def _prep_sc(src, dst, rewards, mu_pad):
    return pl.kernel(
        _prep_body,
        out_type=(jax.ShapeDtypeStruct((E,), jnp.float32),
                  jax.ShapeDtypeStruct((E,), jnp.float32)),
        mesh=_mesh,
        compiler_params=_sc_params,
        name="sc_prep",
        scratch_types=[
            pltpu.VMEM((NPAD,), jnp.float32),
            pltpu.VMEM((CH,), jnp.int32),
            pltpu.VMEM((CH,), jnp.int32),
            pltpu.VMEM((CH,), jnp.int32),
            pltpu.VMEM((CH,), jnp.int32),
            pltpu.VMEM((CH,), jnp.float32),
            pltpu.VMEM((CH,), jnp.float32),
            pltpu.VMEM((CH,), jnp.float32),
            pltpu.VMEM((CH,), jnp.float32),
            pltpu.VMEM((CH,), jnp.float32),
            pltpu.VMEM((CH,), jnp.float32),
            pltpu.SemaphoreType.DMA,
            pltpu.SemaphoreType.DMA,
            pltpu.SemaphoreType.DMA,
            pltpu.SemaphoreType.DMA,
        ],
    )(src, dst, rewards, mu_pad)


def _iter_body(p_hbm, b_hbm, src_hbm, dst_hbm, m_hbm, pw_hbm, pout_hbm,
               x_l, s0, s1, d0, d1, m0, m1, q0, q1, g0, g1, e0, e1,
               acc, sia, sib, ssa, ssb):
    cid = lax.axis_index("c")
    sid = lax.axis_index("s")
    wid = cid * NSUB + sid
    def zvec(j, _):
        g0[pl.ds(j * 16, 16)] = jnp.zeros((16,), jnp.float32)
        return 0

    lax.fori_loop(0, SCH // 16, zvec, 0)
    for part in range(4):
        pltpu.sync_copy(g0.at[pl.ds(0, SCH)],
                        acc.at[pl.ds(sid * SLICE + part * SCH, SCH)])
    _stage_x(p_hbm, b_hbm, x_l, (m0, q0, g0), (m1, q1, g1), sia, sib,
             62, ((97216, 1568), (98784, 1216)))
    plsc.subcore_barrier()

    ebase = wid * EPT
    ins = (src_hbm, dst_hbm, m_hbm, pw_hbm)

    def compute(sb, db, mb, qb, gb, eb):
        @plsc.parallel_loop(0, VPC, 1, unroll=5)
        def vec(i):
            sl = pl.ds(i * 16, 16)
            xs = plsc.load_gather(x_l, [sb[sl]])
            gb[sl] = mb[sl] * _vpow(xs, qb[sl])
            eb[sl] = db[sl]

    _in_start(ins, (s0, d0, m0, q0), ebase, CH, sia)
    _in_start(ins, (s1, d1, m1, q1), ebase + CH, CH, sib)

    def body(k, _):
        c0 = 2 * k
        c1 = c0 + 1
        o0 = ebase + c0 * CH
        o1 = ebase + c1 * CH
        _in_wait(ins, (s0, d0, m0, q0), o0, CH, sia)
        compute(s0, d0, m0, q0, g0, e0)

        @pl.when(c0 >= 2)
        def _():
            pltpu.make_async_copy(g0, acc.at[e0], ssa).wait()

        pltpu.async_copy(g0, acc.at[e0], ssa, add=True)

        @pl.when(c0 + 2 < NCH)
        def _():
            _in_start(ins, (s0, d0, m0, q0), o0 + 2 * CH, CH, sia)

        _in_wait(ins, (s1, d1, m1, q1), o1, CH, sib)
        compute(s1, d1, m1, q1, g1, e1)

        @pl.when(c1 >= 2)
        def _():
            pltpu.make_async_copy(g1, acc.at[e1], ssb).wait()

        pltpu.async_copy(g1, acc.at[e1], ssb, add=True)

        @pl.when(c1 + 2 < NCH)
        def _():
            _in_start(ins, (s1, d1, m1, q1), o1 + 2 * CH, CH, sib)

        return 0

    lax.fori_loop(0, NCH // 2, body, 0)
    pltpu.make_async_copy(g0, acc.at[e0], ssa).wait()
    pltpu.make_async_copy(g1, acc.at[e1], ssb).wait()
    plsc.subcore_barrier()
    for part in range(4):
        pltpu.sync_copy(acc.at[pl.ds(sid * SLICE + part * SCH, SCH)],
                        g0.at[pl.ds(0, SCH)])
        pltpu.sync_copy(
            g0.at[pl.ds(0, SCH)],
            pout_hbm.at[pl.ds(cid * NPAD + sid * SLICE + part * SCH, SCH)])


def _iter_sc(p_prev, b_pad, src, dst, M, pw):
    return pl.kernel(
        _iter_body,
        out_type=jax.ShapeDtypeStruct((2 * NPAD,), jnp.float32),
        mesh=_mesh,
        compiler_params=_sc_params,
        name="sc_iter",
        scratch_types=[
            pltpu.VMEM((N,), jnp.float32),
            pltpu.VMEM((CH,), jnp.int32),
            pltpu.VMEM((CH,), jnp.int32),
            pltpu.VMEM((CH,), jnp.int32),
            pltpu.VMEM((CH,), jnp.int32),
            pltpu.VMEM((CH,), jnp.float32),
            pltpu.VMEM((CH,), jnp.float32),
            pltpu.VMEM((CH,), jnp.float32),
            pltpu.VMEM((CH,), jnp.float32),
            pltpu.VMEM((CH,), jnp.float32),
            pltpu.VMEM((CH,), jnp.float32),
            pltpu.VMEM((CH,), jnp.int32),
            pltpu.VMEM((CH,), jnp.int32),
            pltpu.VMEM_SHARED((NPAD,), jnp.float32),
            pltpu.SemaphoreType.DMA,
            pltpu.SemaphoreType.DMA,
            pltpu.SemaphoreType.DMA,
            pltpu.SemaphoreType.DMA,
        ],
    )(p_prev, b_pad, src, dst, M, pw)


def _epi_body(p_hbm, b_hbm, src_hbm, dst_hbm, m_hbm, pw_hbm,
              val_hbm, ep_hbm,
              x_l, s0, s1, d0, d1, m0, m1, q0, q1, g0, g1, vbuf,
              sia, sib, soa, sob):
    cid = lax.axis_index("c")
    sid = lax.axis_index("s")
    wid = cid * NSUB + sid
    _stage_x(p_hbm, b_hbm, x_l, (m0, q0, g0), (m1, q1, g1), sia, sib,
             NSCH, ())

    vbase = wid * VSLICE

    @plsc.parallel_loop(0, VSLICE // 16, 1, unroll=4)
    def vvec(j):
        xv = x_l[pl.ds(vbase + j * 16, 16)]
        lv = _vlog(xv)
        lv = jnp.where(xv == 0.0,
                       jnp.full((16,), -jnp.inf, jnp.float32), lv)
        vbuf[pl.ds(j * 16, 16)] = lv
    pltpu.sync_copy(vbuf, val_hbm.at[pl.ds(vbase, VSLICE)])

    ebase = wid * EPT
    ins = (src_hbm, dst_hbm, m_hbm, pw_hbm)

    def compute(sb, db, mb, qb, gb):
        @plsc.parallel_loop(0, VPC, 1, unroll=5)
        def vec(i):
            sl = pl.ds(i * 16, 16)
            xs = plsc.load_gather(x_l, [sb[sl]])
            xd = plsc.load_gather(x_l, [db[sl]])
            gb[sl] = mb[sl] * _vpow(xs, qb[sl]) / xd

    _in_start(ins, (s0, d0, m0, q0), ebase, CH, sia)
    _in_start(ins, (s1, d1, m1, q1), ebase + CH, CH, sib)

    def body(k, _):
        c0 = 2 * k
        c1 = c0 + 1
        o0 = ebase + c0 * CH
        o1 = ebase + c1 * CH
        _in_wait(ins, (s0, d0, m0, q0), o0, CH, sia)

        @pl.when(c0 >= 2)
        def _():
            _out_wait((g0,), (ep_hbm,), o0 - 2 * CH, CH, soa)

        compute(s0, d0, m0, q0, g0)
        _out_start((g0,), (ep_hbm,), o0, CH, soa)

        @pl.when(c0 + 2 < NCH)
        def _():
            _in_start(ins, (s0, d0, m0, q0), o0 + 2 * CH, CH, sia)

        _in_wait(ins, (s1, d1, m1, q1), o1, CH, sib)

        @pl.when(c1 >= 2)
        def _():
            _out_wait((g1,), (ep_hbm,), o1 - 2 * CH, CH, sob)

        compute(s1, d1, m1, q1, g1)
        _out_start((g1,), (ep_hbm,), o1, CH, sob)

        @pl.when(c1 + 2 < NCH)
        def _():
            _in_start(ins, (s1, d1, m1, q1), o1 + 2 * CH, CH, sib)

        return 0

    lax.fori_loop(0, NCH // 2, body, 0)
    _out_wait((g0,), (ep_hbm,), ebase + (NCH - 2) * CH, CH, soa)
    _out_wait((g1,), (ep_hbm,), ebase + (NCH - 1) * CH, CH, sob)


def _epi_sc(p_last, b_pad, src, dst, M, pw):
    return pl.kernel(
        _epi_body,
        out_type=(jax.ShapeDtypeStruct((NPAD,), jnp.float32),
                  jax.ShapeDtypeStruct((E,), jnp.float32)),
        mesh=_mesh,
        compiler_params=_sc_params,
        name="sc_epi",
        scratch_types=[
            pltpu.VMEM((NPAD,), jnp.float32),
            pltpu.VMEM((CH,), jnp.int32),
            pltpu.VMEM((CH,), jnp.int32),
            pltpu.VMEM((CH,), jnp.int32),
            pltpu.VMEM((CH,), jnp.int32),
            pltpu.VMEM((CH,), jnp.float32),
            pltpu.VMEM((CH,), jnp.float32),
            pltpu.VMEM((CH,), jnp.float32),
            pltpu.VMEM((CH,), jnp.float32),
            pltpu.VMEM((CH,), jnp.float32),
            pltpu.VMEM((CH,), jnp.float32),
            pltpu.VMEM((VSLICE,), jnp.float32),
            pltpu.SemaphoreType.DMA,
            pltpu.SemaphoreType.DMA,
            pltpu.SemaphoreType.DMA,
            pltpu.SemaphoreType.DMA,
        ],
    )(p_last, b_pad, src, dst, M, pw)


def kernel(edge_index, edge_feats, node_scales, sink_node_mask, W, b0):
    src, dst = _split_tc(edge_index.astype(jnp.int32))
    pad = NPAD - N
    mu_pad = jnp.concatenate(
        [node_scales, jnp.ones((pad,), jnp.float32)])
    b_pad = jnp.concatenate(
        [sink_node_mask, jnp.zeros((pad,), jnp.float32)])

    ef2 = edge_feats.reshape(E // 8, 128)
    wb = jnp.kron(jnp.eye(8, dtype=jnp.float32), W)
    rewards = _rewards_tc(ef2, wb, b0)
    M, pw = _prep_sc(src, dst, rewards, mu_pad)

    p0 = jnp.zeros((2 * NPAD,), jnp.float32)
    p_last = lax.fori_loop(
        0, N_ITERS,
        lambda i, p: _iter_sc(p, b_pad, src, dst, M, pw),
        p0)

    values_pad, edge_probs = _epi_sc(p_last, b_pad, src, dst, M, pw)
    return rewards, values_pad[:N], edge_probs


# speedup vs baseline: 183.0292x; 1.0395x over previous
import jax
import jax.numpy as jnp
from jax import lax
from jax.experimental import pallas as pl
from jax.experimental.pallas import tpu as pltpu
from jax.experimental.pallas import tpu_sc as plsc

N = 100000
E = 3200000
N_ITERS = 12

NCORES = 2
NSUB = 16
NTILES = NCORES * NSUB
LN2 = 0.6931471805599453

NPAD = 100352
SLICE = NPAD // NSUB
VSLICE = NPAD // NTILES
SCH = 1568
NSCH = NPAD // SCH
EPT = E // NTILES
CH = 2000
NCH = EPT // CH
VPC = CH // 16

_mesh = plsc.VectorSubcoreMesh(core_axis_name="c", subcore_axis_name="s")
_sc_params = pltpu.CompilerParams(needs_layout_passes=False)


def _vlog(xv):
    xv = jnp.maximum(xv, jnp.float32(1e-12))
    bits = plsc.bitcast(xv, jnp.int32)
    e = (bits >> 23) - 127
    m = plsc.bitcast((bits & 0x7FFFFF) | 0x3F800000, jnp.float32)
    big = m > jnp.float32(1.4142135)
    m = jnp.where(big, m * jnp.float32(0.5), m)
    ef = (e + big.astype(jnp.int32)).astype(jnp.float32)
    s = (m - 1.0) / (m + 1.0)
    s2 = s * s
    p = jnp.float32(2.0 / 9.0)
    p = p * s2 + jnp.float32(2.0 / 7.0)
    p = p * s2 + jnp.float32(2.0 / 5.0)
    p = p * s2 + jnp.float32(2.0 / 3.0)
    p = p * s2 + jnp.float32(2.0)
    return ef * jnp.float32(LN2) + p * s


def _vpow(xv, pv):
    return jnp.exp(pv * _vlog(xv))


def _rewards_body(ef_ref, wb_ref, b0_ref, out_ref):
    enc = jnp.dot(ef_ref[...], wb_ref[...],
                  preferred_element_type=jnp.float32) + b0_ref[0]
    out_ref[...] = (-jnp.logaddexp(enc, 0.0)).T


def _rewards_tc(ef2, wb, b0):
    rows = E // 8
    blk = 3200
    return pl.pallas_call(
        _rewards_body,
        grid=(rows // blk,),
        in_specs=[
            pl.BlockSpec((blk, 128), lambda i: (i, 0)),
            pl.BlockSpec((128, 8), lambda i: (0, 0)),
            pl.BlockSpec(memory_space=pltpu.SMEM),
        ],
        out_specs=pl.BlockSpec((8, blk), lambda i: (0, i)),
        out_shape=jax.ShapeDtypeStruct((8, rows), jnp.float32),
        compiler_params=pltpu.CompilerParams(
            dimension_semantics=("arbitrary",)),
        name="tc_rewards",
    )(ef2, wb, b0)


def _split_body(ei_ref, src_ref, dst_ref):
    src_ref[...] = ei_ref[0, :]
    dst_ref[...] = ei_ref[1, :]


def _split_tc(edge_index):
    blk = 128000
    return pl.pallas_call(
        _split_body,
        grid=(E // blk,),
        in_specs=[pl.BlockSpec((2, blk), lambda i: (0, i))],
        out_specs=(pl.BlockSpec((blk,), lambda i: (i,)),
                   pl.BlockSpec((blk,), lambda i: (i,))),
        out_shape=(jax.ShapeDtypeStruct((E,), jnp.int32),
                   jax.ShapeDtypeStruct((E,), jnp.int32)),
        compiler_params=pltpu.CompilerParams(
            dimension_semantics=("arbitrary",)),
        name="tc_split",
    )(edge_index)


def _in_start(hbm_refs, bufs, off, n, sem):
    for h, b in zip(hbm_refs, bufs):
        pltpu.async_copy(h.at[pl.ds(off, n)], b.at[pl.ds(0, n)], sem)


def _in_wait(hbm_refs, bufs, off, n, sem):
    for h, b in zip(hbm_refs, bufs):
        pltpu.make_async_copy(h.at[pl.ds(off, n)], b.at[pl.ds(0, n)],
                              sem).wait()


def _out_start(bufs, hbm_refs, off, n, sem):
    for b, h in zip(bufs, hbm_refs):
        pltpu.async_copy(b.at[pl.ds(0, n)], h.at[pl.ds(off, n)], sem)


def _out_wait(bufs, hbm_refs, off, n, sem):
    for b, h in zip(bufs, hbm_refs):
        pltpu.make_async_copy(b.at[pl.ds(0, n)], h.at[pl.ds(off, n)],
                              sem).wait()


def _stage_x(p_hbm, b_hbm, x_l, seta, setb, sema, semb, nring, tails):
    def srcs(c):
        off = c * SCH
        return (p_hbm.at[pl.ds(off, SCH)],
                p_hbm.at[pl.ds(NPAD + off, SCH)],
                b_hbm.at[pl.ds(off, SCH)])

    def start(c, bufs, sem):
        for h, b in zip(srcs(c), bufs):
            pltpu.async_copy(h, b.at[pl.ds(0, SCH)], sem)

    def wait(c, bufs, sem):
        for h, b in zip(srcs(c), bufs):
            pltpu.make_async_copy(h, b.at[pl.ds(0, SCH)], sem).wait()

    def accum(c, bufs):
        fa, fb, fc = bufs

        @plsc.parallel_loop(0, SCH // 16, 1, unroll=7)
        def vec(j):
            sl = pl.ds(j * 16, 16)
            x_l[pl.ds(c * SCH + j * 16, 16)] = fa[sl] + fb[sl] + fc[sl]

    start(0, seta, sema)
    start(1, setb, semb)

    def body(k, _):
        c0 = 2 * k
        c1 = c0 + 1
        wait(c0, seta, sema)
        accum(c0, seta)

        @pl.when(c0 + 2 < nring)
        def _():
            start(c0 + 2, seta, sema)

        wait(c1, setb, semb)
        accum(c1, setb)

        @pl.when(c1 + 2 < nring)
        def _():
            start(c1 + 2, setb, semb)

        return 0

    lax.fori_loop(0, nring // 2, body, 0)
    fa, fb, fc = seta
    for off, sz in tails:
        pltpu.sync_copy(p_hbm.at[pl.ds(off, sz)], fa.at[pl.ds(0, sz)])
        pltpu.sync_copy(p_hbm.at[pl.ds(NPAD + off, sz)], fb.at[pl.ds(0, sz)])
        pltpu.sync_copy(b_hbm.at[pl.ds(off, sz)], fc.at[pl.ds(0, sz)])

        def tvec(j, _, off=off):
            sl = pl.ds(j * 16, 16)
            x_l[pl.ds(off + j * 16, 16)] = fa[sl] + fb[sl] + fc[sl]
            return 0

        lax.fori_loop(0, sz // 16, tvec, 0)


def _prep_body(src_hbm, dst_hbm, rew_hbm, mu_hbm, m_hbm, pw_hbm, ro_hbm,
               mu_l, s0, s1, d0, d1, r0, r1, m0, m1, q0, q1, ob0, ob1,
               sia, sib, soa, sob):
    cid = lax.axis_index("c")
    sid = lax.axis_index("s")
    wid = cid * NSUB + sid
    pltpu.sync_copy(mu_hbm, mu_l)
    ebase = wid * EPT
    ins = (src_hbm, dst_hbm)
    outs = (m_hbm, pw_hbm, ro_hbm)

    def rew_base(off):
        row0 = off >> 3
        base = pl.multiple_of((row0 >> 3) << 3, 8)
        return base, row0 - base

    def rew_start(off, rb, sem):
        base, _ = rew_base(off)
        for c in range(8):
            pltpu.async_copy(rew_hbm.at[pl.ds(c * (E // 8) + base, 256)],
                             rb.at[pl.ds(c * 256, 256)], sem)

    def rew_wait(off, rb, sem):
        base, _ = rew_base(off)
        for c in range(8):
            pltpu.make_async_copy(
                rew_hbm.at[pl.ds(c * (E // 8) + base, 256)],
                rb.at[pl.ds(c * 256, 256)], sem).wait()

    def compute(off, sb, db, rb, mb, qb, rob):
        _, delta = rew_base(off)

        @plsc.parallel_loop(0, VPC, 1, unroll=5)
        def vec(i):
            sl = pl.ds(i * 16, 16)
            k = lax.broadcasted_iota(jnp.int32, (16,), 0)
            ridx = (k & 7) * 256 + (k >> 3) + (2 * i + delta)
            rv = plsc.load_gather(rb, [ridx])
            mu_j = plsc.load_gather(mu_l, [sb[sl]])
            mu_i = plsc.load_gather(mu_l, [db[sl]])
            inv = 1.0 / mu_i
            mb[sl] = jnp.exp(rv * inv)
            qb[sl] = mu_j * inv
            rob[sl] = rv

    _in_start(ins, (s0, d0), ebase, CH, sia)
    rew_start(ebase, r0, sia)
    _in_start(ins, (s1, d1), ebase + CH, CH, sib)
    rew_start(ebase + CH, r1, sib)

    def body(k, _):
        c0 = 2 * k
        c1 = c0 + 1
        o0 = ebase + c0 * CH
        o1 = ebase + c1 * CH
        _in_wait(ins, (s0, d0), o0, CH, sia)
        rew_wait(o0, r0, sia)

        @pl.when(c0 >= 2)
        def _():
            _out_wait((m0, q0, ob0), outs, o0 - 2 * CH, CH, soa)

        compute(o0, s0, d0, r0, m0, q0, ob0)
        _out_start((m0, q0, ob0), outs, o0, CH, soa)

        @pl.when(c0 + 2 < NCH)
        def _():
            _in_start(ins, (s0, d0), o0 + 2 * CH, CH, sia)
            rew_start(o0 + 2 * CH, r0, sia)

        _in_wait(ins, (s1, d1), o1, CH, sib)
        rew_wait(o1, r1, sib)

        @pl.when(c1 >= 2)
        def _():
            _out_wait((m1, q1, ob1), outs, o1 - 2 * CH, CH, sob)

        compute(o1, s1, d1, r1, m1, q1, ob1)
        _out_start((m1, q1, ob1), outs, o1, CH, sob)

        @pl.when(c1 + 2 < NCH)
        def _():
            _in_start(ins, (s1, d1), o1 + 2 * CH, CH, sib)
            rew_start(o1 + 2 * CH, r1, sib)

        return 0

    lax.fori_loop(0, NCH // 2, body, 0)
    _out_wait((m0, q0, ob0), outs, ebase + (NCH - 2) * CH, CH, soa)
    _out_wait((m1, q1, ob1), outs, ebase + (NCH - 1) * CH, CH, sob)


def _prep_sc(src, dst, rew_planes, mu_pad):
    return pl.kernel(
        _prep_body,
        out_type=(jax.ShapeDtypeStruct((E,), jnp.float32),
                  jax.ShapeDtypeStruct((E,), jnp.float32),
                  jax.ShapeDtypeStruct((E,), jnp.float32)),
        mesh=_mesh,
        compiler_params=_sc_params,
        name="sc_prep",
        scratch_types=[
            pltpu.VMEM((NPAD,), jnp.float32),
            pltpu.VMEM((CH,), jnp.int32),
            pltpu.VMEM((CH,), jnp.int32),
            pltpu.VMEM((CH,), jnp.int32),
            pltpu.VMEM((CH,), jnp.int32),
            pltpu.VMEM((2048,), jnp.float32),
            pltpu.VMEM((2048,), jnp.float32),
            pltpu.VMEM((CH,), jnp.float32),
            pltpu.VMEM((CH,), jnp.float32),
            pltpu.VMEM((CH,), jnp.float32),
            pltpu.VMEM((CH,), jnp.float32),
            pltpu.VMEM((CH,), jnp.float32),
            pltpu.VMEM((CH,), jnp.float32),
            pltpu.SemaphoreType.DMA,
            pltpu.SemaphoreType.DMA,
            pltpu.SemaphoreType.DMA,
            pltpu.SemaphoreType.DMA,
        ],
    )(src, dst, rew_planes, mu_pad)


def _iter_body(p_hbm, b_hbm, src_hbm, dst_hbm, m_hbm, pw_hbm, pout_hbm,
               x_l, s0, s1, d0, d1, m0, m1, q0, q1, g0, g1, e0, e1,
               acc, sia, sib, ssa, ssb):
    cid = lax.axis_index("c")
    sid = lax.axis_index("s")
    wid = cid * NSUB + sid
    def zvec(j, _):
        g0[pl.ds(j * 16, 16)] = jnp.zeros((16,), jnp.float32)
        return 0

    lax.fori_loop(0, SCH // 16, zvec, 0)
    for part in range(4):
        pltpu.sync_copy(g0.at[pl.ds(0, SCH)],
                        acc.at[pl.ds(sid * SLICE + part * SCH, SCH)])
    _stage_x(p_hbm, b_hbm, x_l, (m0, q0, g0), (m1, q1, g1), sia, sib,
             62, ((97216, 1568), (98784, 1216)))
    plsc.subcore_barrier()

    ebase = wid * EPT
    ins = (src_hbm, dst_hbm, m_hbm, pw_hbm)

    def compute(sb, db, mb, qb, gb, eb):
        @plsc.parallel_loop(0, VPC, 1, unroll=5)
        def vec(i):
            sl = pl.ds(i * 16, 16)
            xs = plsc.load_gather(x_l, [sb[sl]])
            gb[sl] = mb[sl] * _vpow(xs, qb[sl])
            eb[sl] = db[sl]

    _in_start(ins, (s0, d0, m0, q0), ebase, CH, sia)
    _in_start(ins, (s1, d1, m1, q1), ebase + CH, CH, sib)

    def body(k, _):
        c0 = 2 * k
        c1 = c0 + 1
        o0 = ebase + c0 * CH
        o1 = ebase + c1 * CH
        _in_wait(ins, (s0, d0, m0, q0), o0, CH, sia)
        compute(s0, d0, m0, q0, g0, e0)

        @pl.when(c0 >= 2)
        def _():
            pltpu.make_async_copy(g0, acc.at[e0], ssa).wait()

        pltpu.async_copy(g0, acc.at[e0], ssa, add=True)

        @pl.when(c0 + 2 < NCH)
        def _():
            _in_start(ins, (s0, d0, m0, q0), o0 + 2 * CH, CH, sia)

        _in_wait(ins, (s1, d1, m1, q1), o1, CH, sib)
        compute(s1, d1, m1, q1, g1, e1)

        @pl.when(c1 >= 2)
        def _():
            pltpu.make_async_copy(g1, acc.at[e1], ssb).wait()

        pltpu.async_copy(g1, acc.at[e1], ssb, add=True)

        @pl.when(c1 + 2 < NCH)
        def _():
            _in_start(ins, (s1, d1, m1, q1), o1 + 2 * CH, CH, sib)

        return 0

    lax.fori_loop(0, NCH // 2, body, 0)
    pltpu.make_async_copy(g0, acc.at[e0], ssa).wait()
    pltpu.make_async_copy(g1, acc.at[e1], ssb).wait()
    plsc.subcore_barrier()
    for part in range(4):
        pltpu.sync_copy(acc.at[pl.ds(sid * SLICE + part * SCH, SCH)],
                        g0.at[pl.ds(0, SCH)])
        pltpu.sync_copy(
            g0.at[pl.ds(0, SCH)],
            pout_hbm.at[pl.ds(cid * NPAD + sid * SLICE + part * SCH, SCH)])


def _iter_sc(p_prev, b_pad, src, dst, M, pw):
    return pl.kernel(
        _iter_body,
        out_type=jax.ShapeDtypeStruct((2 * NPAD,), jnp.float32),
        mesh=_mesh,
        compiler_params=_sc_params,
        name="sc_iter",
        scratch_types=[
            pltpu.VMEM((N,), jnp.float32),
            pltpu.VMEM((CH,), jnp.int32),
            pltpu.VMEM((CH,), jnp.int32),
            pltpu.VMEM((CH,), jnp.int32),
            pltpu.VMEM((CH,), jnp.int32),
            pltpu.VMEM((CH,), jnp.float32),
            pltpu.VMEM((CH,), jnp.float32),
            pltpu.VMEM((CH,), jnp.float32),
            pltpu.VMEM((CH,), jnp.float32),
            pltpu.VMEM((CH,), jnp.float32),
            pltpu.VMEM((CH,), jnp.float32),
            pltpu.VMEM((CH,), jnp.int32),
            pltpu.VMEM((CH,), jnp.int32),
            pltpu.VMEM_SHARED((NPAD,), jnp.float32),
            pltpu.SemaphoreType.DMA,
            pltpu.SemaphoreType.DMA,
            pltpu.SemaphoreType.DMA,
            pltpu.SemaphoreType.DMA,
        ],
    )(p_prev, b_pad, src, dst, M, pw)


def _epi_body(p_hbm, b_hbm, src_hbm, dst_hbm, m_hbm, pw_hbm,
              val_hbm, ep_hbm,
              x_l, s0, s1, d0, d1, m0, m1, q0, q1, g0, g1, vbuf,
              sia, sib, soa, sob):
    cid = lax.axis_index("c")
    sid = lax.axis_index("s")
    wid = cid * NSUB + sid
    _stage_x(p_hbm, b_hbm, x_l, (m0, q0, g0), (m1, q1, g1), sia, sib,
             NSCH, ())

    vbase = wid * VSLICE

    @plsc.parallel_loop(0, VSLICE // 16, 1, unroll=4)
    def vvec(j):
        xv = x_l[pl.ds(vbase + j * 16, 16)]
        lv = _vlog(xv)
        lv = jnp.where(xv == 0.0,
                       jnp.full((16,), -jnp.inf, jnp.float32), lv)
        vbuf[pl.ds(j * 16, 16)] = lv
    pltpu.sync_copy(vbuf, val_hbm.at[pl.ds(vbase, VSLICE)])

    ebase = wid * EPT
    ins = (src_hbm, dst_hbm, m_hbm, pw_hbm)

    def compute(sb, db, mb, qb, gb):
        @plsc.parallel_loop(0, VPC, 1, unroll=5)
        def vec(i):
            sl = pl.ds(i * 16, 16)
            xs = plsc.load_gather(x_l, [sb[sl]])
            xd = plsc.load_gather(x_l, [db[sl]])
            gb[sl] = mb[sl] * _vpow(xs, qb[sl]) / xd

    _in_start(ins, (s0, d0, m0, q0), ebase, CH, sia)
    _in_start(ins, (s1, d1, m1, q1), ebase + CH, CH, sib)

    def body(k, _):
        c0 = 2 * k
        c1 = c0 + 1
        o0 = ebase + c0 * CH
        o1 = ebase + c1 * CH
        _in_wait(ins, (s0, d0, m0, q0), o0, CH, sia)

        @pl.when(c0 >= 2)
        def _():
            _out_wait((g0,), (ep_hbm,), o0 - 2 * CH, CH, soa)

        compute(s0, d0, m0, q0, g0)
        _out_start((g0,), (ep_hbm,), o0, CH, soa)

        @pl.when(c0 + 2 < NCH)
        def _():
            _in_start(ins, (s0, d0, m0, q0), o0 + 2 * CH, CH, sia)

        _in_wait(ins, (s1, d1, m1, q1), o1, CH, sib)

        @pl.when(c1 >= 2)
        def _():
            _out_wait((g1,), (ep_hbm,), o1 - 2 * CH, CH, sob)

        compute(s1, d1, m1, q1, g1)
        _out_start((g1,), (ep_hbm,), o1, CH, sob)

        @pl.when(c1 + 2 < NCH)
        def _():
            _in_start(ins, (s1, d1, m1, q1), o1 + 2 * CH, CH, sib)

        return 0

    lax.fori_loop(0, NCH // 2, body, 0)
    _out_wait((g0,), (ep_hbm,), ebase + (NCH - 2) * CH, CH, soa)
    _out_wait((g1,), (ep_hbm,), ebase + (NCH - 1) * CH, CH, sob)


def _epi_sc(p_last, b_pad, src, dst, M, pw):
    return pl.kernel(
        _epi_body,
        out_type=(jax.ShapeDtypeStruct((NPAD,), jnp.float32),
                  jax.ShapeDtypeStruct((E,), jnp.float32)),
        mesh=_mesh,
        compiler_params=_sc_params,
        name="sc_epi",
        scratch_types=[
            pltpu.VMEM((NPAD,), jnp.float32),
            pltpu.VMEM((CH,), jnp.int32),
            pltpu.VMEM((CH,), jnp.int32),
            pltpu.VMEM((CH,), jnp.int32),
            pltpu.VMEM((CH,), jnp.int32),
            pltpu.VMEM((CH,), jnp.float32),
            pltpu.VMEM((CH,), jnp.float32),
            pltpu.VMEM((CH,), jnp.float32),
            pltpu.VMEM((CH,), jnp.float32),
            pltpu.VMEM((CH,), jnp.float32),
            pltpu.VMEM((CH,), jnp.float32),
            pltpu.VMEM((VSLICE,), jnp.float32),
            pltpu.SemaphoreType.DMA,
            pltpu.SemaphoreType.DMA,
            pltpu.SemaphoreType.DMA,
            pltpu.SemaphoreType.DMA,
        ],
    )(p_last, b_pad, src, dst, M, pw)


def kernel(edge_index, edge_feats, node_scales, sink_node_mask, W, b0):
    src, dst = _split_tc(edge_index.astype(jnp.int32))
    pad = NPAD - N
    mu_pad = jnp.concatenate(
        [node_scales, jnp.ones((pad,), jnp.float32)])
    b_pad = jnp.concatenate(
        [sink_node_mask, jnp.zeros((pad,), jnp.float32)])

    ef2 = edge_feats.reshape(E // 8, 128)
    wb = jnp.kron(jnp.eye(8, dtype=jnp.float32), W)
    rew_planes = _rewards_tc(ef2, wb, b0).reshape(E)
    M, pw, rewards = _prep_sc(src, dst, rew_planes, mu_pad)

    p0 = jnp.zeros((2 * NPAD,), jnp.float32)
    p_last = lax.fori_loop(
        0, N_ITERS,
        lambda i, p: _iter_sc(p, b_pad, src, dst, M, pw),
        p0)

    values_pad, edge_probs = _epi_sc(p_last, b_pad, src, dst, M, pw)
    return rewards, values_pad[:N], edge_probs
